# Initial kernel scaffold; baseline (speedup 1.0000x reference)
import jax
import jax.numpy as jnp
from jax.experimental import pallas as pl


def kernel(x, edge_index, batch, W_in, b_in, W_h0, b_h0, W_h1, b_h1, W_fc, b_fc):
    raise NotImplementedError("write your pallas kernel here")


# speedup vs baseline: 13.4184x; 13.4184x over previous
import functools

import jax
import jax.numpy as jnp
from jax import lax
from jax.experimental import pallas as pl
from jax.experimental.pallas import tpu as pltpu
from jax.experimental.pallas import tpu_sc as plsc

N = 10000
NPAD = 10240
BIN = N
G = 64
D = 128
W128 = 128
E_LOOP = 320000 + N
EP = 331776
NW = 32
CHK = 81
B = 128
RPT = NPAD // NW
RPS = NPAD // 16
DEGW = 128

_mesh = plsc.VectorSubcoreMesh(core_axis_name="c", subcore_axis_name="s")


def _wid():
    return lax.axis_index("s") * 2 + lax.axis_index("c")


@functools.partial(
    pl.kernel,
    out_type=jax.ShapeDtypeStruct((2, NPAD, DEGW), jnp.float32),
    mesh=_mesh,
    scratch_types=[
        pltpu.VMEM_SHARED((NPAD, DEGW), jnp.float32),
        pltpu.VMEM((8, DEGW), jnp.float32),
        pltpu.VMEM((B, DEGW), jnp.float32),
        pltpu.VMEM((CHK, B), jnp.int32),
    ],
)
def _k_deg(dstblk, degpart, acc, zbuf, ones_v, idx_v):
    c = lax.axis_index("c")
    s = lax.axis_index("s")
    wid = _wid()
    z = jnp.zeros((16,), jnp.float32)
    one = jnp.ones((16,), jnp.float32)

    def zb(i, _):
        zbuf[i // 8, pl.ds((i % 8) * 16, 16)] = z
        return 0

    lax.fori_loop(0, 8 * (DEGW // 16), zb, 0)

    def fo(i, _):
        ones_v[i // 8, pl.ds((i % 8) * 16, 16)] = one
        return 0

    lax.fori_loop(0, B * (DEGW // 16), fo, 0)

    def zacc(i, _):
        pltpu.sync_copy(zbuf, acc.at[pl.ds(s * RPS + i * 8, 8)])
        return 0

    lax.fori_loop(0, RPS // 8, zacc, 0)
    plsc.subcore_barrier()
    pltpu.sync_copy(dstblk.at[wid], idx_v)

    def chunk(j, _):
        pltpu.sync_copy(ones_v, acc.at[idx_v.at[j]], add=True)
        return 0

    lax.fori_loop(0, CHK, chunk, 0)
    plsc.subcore_barrier()
    pltpu.sync_copy(acc.at[pl.ds(s * RPS, RPS)],
                    degpart.at[c].at[pl.ds(s * RPS, RPS)])


@functools.partial(
    pl.kernel,
    out_type=jax.ShapeDtypeStruct((NPAD * W128,), jnp.float32),
    mesh=_mesh,
    scratch_types=[
        pltpu.VMEM((RPT * W128,), jnp.float32),
        pltpu.VMEM((RPT,), jnp.int32),
    ],
)
def _k_onehot(batchp, y0, loc_y, batch_v):
    wid = _wid()
    pltpu.sync_copy(batchp.at[pl.ds(wid * RPT, RPT)], batch_v)
    it = lax.broadcasted_iota(jnp.int32, (16,), 0)
    gdn = lax.GatherDimensionNumbers(
        offset_dims=(), collapsed_slice_dims=(0,), start_index_map=(0,))

    def grp(g, _):
        bvv = batch_v[pl.ds(g * 16, 16)]
        for e in range(16):
            idx = jnp.full((16, 1), e, jnp.int32)
            bc = lax.gather(bvv, idx, gdn, slice_sizes=(1,),
                            mode=lax.GatherScatterMode.PROMISE_IN_BOUNDS)
            for kk in range(W128 // 16):
                row = jnp.where(it + kk * 16 == bc, 1.0, 0.0)
                loc_y[pl.ds((g * 16 + e) * W128 + kk * 16, 16)] = row
        return 0

    lax.fori_loop(0, RPT // 16, grp, 0)
    pltpu.sync_copy(loc_y, y0.at[pl.ds(wid * RPT * W128, RPT * W128)])


@functools.partial(
    pl.kernel,
    out_type=jax.ShapeDtypeStruct((2, NPAD, W128), jnp.float32),
    mesh=_mesh,
    scratch_types=[
        pltpu.VMEM_SHARED((NPAD, W128), jnp.float32),
        pltpu.VMEM((8, W128), jnp.float32),
        pltpu.VMEM((CHK, B), jnp.int32),
        pltpu.VMEM((CHK, B), jnp.int32),
        pltpu.VMEM((B, W128), jnp.float32),
        pltpu.SemaphoreType.DMA,
    ],
)
def _k_pass(yin, srcblk, dstblk, part, acc, zbuf, sv, dv, rows, sem):
    c = lax.axis_index("c")
    s = lax.axis_index("s")
    wid = _wid()
    z = jnp.zeros((16,), jnp.float32)

    def zb(i, _):
        zbuf[i // 8, pl.ds((i % 8) * 16, 16)] = z
        return 0

    lax.fori_loop(0, 8 * 8, zb, 0)

    def zacc(i, _):
        pltpu.sync_copy(zbuf, acc.at[pl.ds(s * RPS + i * 8, 8)])
        return 0

    lax.fori_loop(0, RPS // 8, zacc, 0)
    plsc.subcore_barrier()

    pltpu.sync_copy(srcblk.at[wid], sv)
    pltpu.sync_copy(dstblk.at[wid], dv)

    def chunk(j, _):
        pltpu.async_copy(yin.at[dv.at[j]], rows, sem).wait()
        pltpu.sync_copy(rows, acc.at[sv.at[j]], add=True)
        return 0

    lax.fori_loop(0, CHK, chunk, 0)
    plsc.subcore_barrier()
    pltpu.sync_copy(acc.at[pl.ds(s * RPS, RPS)],
                    part.at[c].at[pl.ds(s * RPS, RPS)])


def _prep_body(dp_ref, y0_ref, y0s_ref, dis_ref, dinv_ref):
    deg = dp_ref[0, :, 0:1] + dp_ref[1, :, 0:1]
    deg = jnp.maximum(deg, 1.0)
    dis = lax.rsqrt(deg)
    dis_ref[...] = dis
    dinv_ref[...] = 1.0 / deg
    y0s_ref[...] = y0_ref[...] * dis


def _prep(degpart, y0):
    return pl.pallas_call(
        _prep_body,
        out_shape=(
            jax.ShapeDtypeStruct((NPAD, W128), jnp.float32),
            jax.ShapeDtypeStruct((NPAD, 1), jnp.float32),
            jax.ShapeDtypeStruct((NPAD, 1), jnp.float32),
        ),
    )(degpart, y0)


def _comb_body(p_ref, dinv_ref, z_ref, ynext_ref):
    zsum = p_ref[0] + p_ref[1]
    z_ref[...] = zsum
    ynext_ref[...] = zsum * dinv_ref[...]


def _comb(part, dinv):
    return pl.pallas_call(
        _comb_body,
        out_shape=(
            jax.ShapeDtypeStruct((NPAD, W128), jnp.float32),
            jax.ShapeDtypeStruct((NPAD, W128), jnp.float32),
        ),
    )(part, dinv)


def _head_body(y0_ref, z1_ref, z2_ref, p3_ref, dis_ref, x_ref, w1_ref, w2_ref,
               w3_ref, wfc_ref, b1_ref, b2_ref, b3_ref, bfc_ref, o_ref):
    f32 = jnp.float32
    hi = jax.lax.Precision.HIGHEST
    dis = dis_ref[...]
    y3 = ((p3_ref[0] + p3_ref[1]) * dis)[:, 0:G]
    dn = (((0,), (0,)), ((), ()))
    q = lax.dot_general(y3, x_ref[...], dn, precision=hi,
                        preferred_element_type=f32)
    ones_col = jnp.ones((NPAD, 1), f32)
    cnt = lax.dot_general(y0_ref[:, 0:G], ones_col, dn, precision=hi,
                          preferred_element_type=f32)
    w1s = lax.dot_general(z1_ref[:, 0:G], dis, dn, precision=hi,
                          preferred_element_type=f32)
    w2s = lax.dot_general(z2_ref[:, 0:G], dis, dn, precision=hi,
                          preferred_element_type=f32)
    mm = functools.partial(lax.dot_general,
                           dimension_numbers=(((1,), (0,)), ((), ())),
                           precision=hi, preferred_element_type=f32)
    t1 = mm(w3_ref[...], wfc_ref[...])
    t2 = mm(w2_ref[...], t1)
    m3 = mm(w1_ref[...], t2)
    c2 = mm(b1_ref[...], t2)
    c1 = mm(b2_ref[...], t1)
    c0 = mm(b3_ref[...], wfc_ref[...])
    raw = mm(q, m3) + mm(w2s, c2) + mm(w1s, c1) + mm(cnt, c0)
    o_ref[...] = raw / jnp.maximum(cnt, 1.0) + bfc_ref[...]


def _head(y0, z1, z2, p3, dis, xpad, W_in, W_h0, W_h1, W_fc,
          b_in, b_h0, b_h1, b_fc):
    return pl.pallas_call(
        _head_body,
        out_shape=jax.ShapeDtypeStruct((G, G), jnp.float32),
    )(y0, z1, z2, p3, dis, xpad, W_in, W_h0, W_h1, W_fc,
      b_in.reshape(1, D), b_h0.reshape(1, D), b_h1.reshape(1, D),
      b_fc.reshape(1, G))


def kernel(x, edge_index, batch, W_in, b_in, W_h0, b_h0, W_h1, b_h1, W_fc, b_fc):
    i32 = jnp.int32
    src = edge_index[0].astype(i32)
    dst = edge_index[1].astype(i32)
    loops = jnp.arange(N, dtype=i32)
    padi = jnp.full((EP - E_LOOP,), BIN, i32)
    s_p = jnp.concatenate([src, loops, padi]).reshape(NW, CHK, B)
    d_p = jnp.concatenate([dst, loops, padi]).reshape(NW, CHK, B)
    bpad = jnp.pad(batch.astype(i32), (0, NPAD - N), constant_values=-1)
    xpad = jnp.pad(x, ((0, NPAD - N), (0, 0)))

    degpart = _k_deg(d_p)
    y0 = _k_onehot(bpad).reshape(NPAD, W128)
    y0s, dis, dinv = _prep(degpart, y0)
    p1 = _k_pass(y0s, s_p, d_p)
    z1, y1s = _comb(p1, dinv)
    p2 = _k_pass(y1s, s_p, d_p)
    z2, y2s = _comb(p2, dinv)
    p3 = _k_pass(y2s, s_p, d_p)
    return _head(y0, z1, z2, p3, dis, xpad, W_in, W_h0, W_h1, W_fc,
                 b_in, b_h0, b_h1, b_fc)


# speedup vs baseline: 15.7945x; 1.1771x over previous
import functools

import jax
import jax.numpy as jnp
from jax import lax
from jax.experimental import pallas as pl
from jax.experimental.pallas import tpu as pltpu
from jax.experimental.pallas import tpu_sc as plsc

N = 10000
NPAD = 10240
BIN = N
G = 64
D = 128
W128 = 128
E_LOOP = 320000 + N
EP = 331776
NW = 32
CHK = 81
B = 128
RPT = NPAD // NW
RPS = NPAD // 16
DEGW = 128

_mesh = plsc.VectorSubcoreMesh(core_axis_name="c", subcore_axis_name="s")


def _wid():
    return lax.axis_index("s") * 2 + lax.axis_index("c")


@functools.partial(
    pl.kernel,
    out_type=jax.ShapeDtypeStruct((2, NPAD, DEGW), jnp.float32),
    mesh=_mesh,
    scratch_types=[
        pltpu.VMEM_SHARED((NPAD, DEGW), jnp.float32),
        pltpu.VMEM((B, DEGW), jnp.float32),
        pltpu.VMEM((B, DEGW), jnp.float32),
        pltpu.VMEM((CHK, B), jnp.int32),
        pltpu.SemaphoreType.DMA,
    ],
)
def _k_deg(dstblk, degpart, acc, zbuf, ones_v, idx_v, sem):
    c = lax.axis_index("c")
    s = lax.axis_index("s")
    wid = _wid()
    z = jnp.zeros((16,), jnp.float32)
    one = jnp.ones((16,), jnp.float32)

    def zb(i, _):
        zbuf[i // 8, pl.ds((i % 8) * 16, 16)] = z
        ones_v[i // 8, pl.ds((i % 8) * 16, 16)] = one
        return 0

    lax.fori_loop(0, B * (DEGW // 16), zb, 0)

    def zacc(i, _):
        pltpu.sync_copy(zbuf, acc.at[pl.ds(s * RPS + i * B, B)])
        return 0

    lax.fori_loop(0, RPS // B, zacc, 0)
    plsc.subcore_barrier()
    pltpu.sync_copy(dstblk.at[wid], idx_v)

    def chunk(j, _):
        pltpu.async_copy(ones_v, acc.at[idx_v.at[j]], sem, add=True)
        return 0

    lax.fori_loop(0, CHK, chunk, 0)

    def drain(j, _):
        pltpu.make_async_copy(ones_v, acc.at[idx_v.at[0]], sem).wait()
        return 0

    lax.fori_loop(0, CHK, drain, 0)
    plsc.subcore_barrier()
    pltpu.sync_copy(acc.at[pl.ds(s * RPS, RPS)],
                    degpart.at[c].at[pl.ds(s * RPS, RPS)])


@functools.partial(
    pl.kernel,
    out_type=jax.ShapeDtypeStruct((NPAD * W128,), jnp.float32),
    mesh=_mesh,
    scratch_types=[
        pltpu.VMEM((RPT * W128,), jnp.float32),
        pltpu.VMEM((RPT,), jnp.int32),
    ],
)
def _k_onehot(batchp, y0, loc_y, batch_v):
    wid = _wid()
    pltpu.sync_copy(batchp.at[pl.ds(wid * RPT, RPT)], batch_v)
    it = lax.broadcasted_iota(jnp.int32, (16,), 0)
    gdn = lax.GatherDimensionNumbers(
        offset_dims=(), collapsed_slice_dims=(0,), start_index_map=(0,))

    def grp(g, _):
        bvv = batch_v[pl.ds(g * 16, 16)]
        for e in range(16):
            idx = jnp.full((16, 1), e, jnp.int32)
            bc = lax.gather(bvv, idx, gdn, slice_sizes=(1,),
                            mode=lax.GatherScatterMode.PROMISE_IN_BOUNDS)
            for kk in range(W128 // 16):
                row = jnp.where(it + kk * 16 == bc, 1.0, 0.0)
                loc_y[pl.ds((g * 16 + e) * W128 + kk * 16, 16)] = row
        return 0

    lax.fori_loop(0, RPT // 16, grp, 0)
    pltpu.sync_copy(loc_y, y0.at[pl.ds(wid * RPT * W128, RPT * W128)])


@functools.partial(
    pl.kernel,
    out_type=jax.ShapeDtypeStruct((2, NPAD, W128), jnp.float32),
    mesh=_mesh,
    scratch_types=[
        pltpu.VMEM_SHARED((NPAD, W128), jnp.float32),
        pltpu.VMEM((CHK, B), jnp.int32),
        pltpu.VMEM((2, B), jnp.int32),
        pltpu.VMEM((B, W128), jnp.float32),
        pltpu.VMEM((B, W128), jnp.float32),
        pltpu.SemaphoreType.DMA,
        pltpu.SemaphoreType.DMA,
    ],
)
def _k_pass(yin, srcblk, dstblk, part, acc, sv, dvb, r0, r1, sg0, sg1):
    c = lax.axis_index("c")
    s = lax.axis_index("s")
    wid = _wid()
    z = jnp.zeros((16,), jnp.float32)

    def zb(i, _):
        r0[i // 8, pl.ds((i % 8) * 16, 16)] = z
        return 0

    lax.fori_loop(0, B * 8, zb, 0)

    def zacc(i, _):
        pltpu.sync_copy(r0, acc.at[pl.ds(s * RPS + i * B, B)])
        return 0

    lax.fori_loop(0, RPS // B, zacc, 0)
    plsc.subcore_barrier()

    pltpu.sync_copy(srcblk.at[wid], sv)
    dvh = dstblk.at[wid]
    pltpu.sync_copy(dvh.at[0], dvb.at[0])
    pltpu.sync_copy(dvh.at[1], dvb.at[1])
    pltpu.async_copy(yin.at[dvb.at[0]], r0, sg0)

    def chunk(j, _):
        par = j & 1

        @pl.when(par == 0)
        def _():
            pltpu.make_async_copy(yin.at[dvb.at[0]], r0, sg0).wait()

            @pl.when(j + 1 < CHK)
            def _():
                pltpu.async_copy(yin.at[dvb.at[1]], r1, sg1)

            @pl.when(j + 2 < CHK)
            def _():
                pltpu.sync_copy(dvh.at[j + 2], dvb.at[0])

            pltpu.sync_copy(r0, acc.at[sv.at[j]], add=True)

        @pl.when(par == 1)
        def _():
            pltpu.make_async_copy(yin.at[dvb.at[1]], r1, sg1).wait()

            @pl.when(j + 1 < CHK)
            def _():
                pltpu.async_copy(yin.at[dvb.at[0]], r0, sg0)

            @pl.when(j + 2 < CHK)
            def _():
                pltpu.sync_copy(dvh.at[j + 2], dvb.at[1])

            pltpu.sync_copy(r1, acc.at[sv.at[j]], add=True)

        return 0

    lax.fori_loop(0, CHK, chunk, 0)
    plsc.subcore_barrier()
    pltpu.sync_copy(acc.at[pl.ds(s * RPS, RPS)],
                    part.at[c].at[pl.ds(s * RPS, RPS)])


def _prep_body(dp_ref, y0_ref, y0s_ref, dis_ref, dinv_ref):
    deg = dp_ref[0, :, 0:1] + dp_ref[1, :, 0:1]
    deg = jnp.maximum(deg, 1.0)
    dis = lax.rsqrt(deg)
    dis_ref[...] = dis
    dinv_ref[...] = 1.0 / deg
    y0s_ref[...] = y0_ref[...] * dis


def _prep(degpart, y0):
    return pl.pallas_call(
        _prep_body,
        out_shape=(
            jax.ShapeDtypeStruct((NPAD, W128), jnp.float32),
            jax.ShapeDtypeStruct((NPAD, 1), jnp.float32),
            jax.ShapeDtypeStruct((NPAD, 1), jnp.float32),
        ),
    )(degpart, y0)


def _comb_body(p_ref, dinv_ref, z_ref, ynext_ref):
    zsum = p_ref[0] + p_ref[1]
    z_ref[...] = zsum
    ynext_ref[...] = zsum * dinv_ref[...]


def _comb(part, dinv):
    return pl.pallas_call(
        _comb_body,
        out_shape=(
            jax.ShapeDtypeStruct((NPAD, W128), jnp.float32),
            jax.ShapeDtypeStruct((NPAD, W128), jnp.float32),
        ),
    )(part, dinv)


def _head_body(y0_ref, z1_ref, z2_ref, p3_ref, dis_ref, x_ref, w1_ref, w2_ref,
               w3_ref, wfc_ref, b1_ref, b2_ref, b3_ref, bfc_ref, o_ref):
    f32 = jnp.float32
    hi = jax.lax.Precision.HIGHEST
    dis = dis_ref[...]
    y3 = ((p3_ref[0] + p3_ref[1]) * dis)[:, 0:G]
    dn = (((0,), (0,)), ((), ()))
    q = lax.dot_general(y3, x_ref[...], dn, precision=hi,
                        preferred_element_type=f32)
    ones_col = jnp.ones((NPAD, 1), f32)
    cnt = lax.dot_general(y0_ref[:, 0:G], ones_col, dn, precision=hi,
                          preferred_element_type=f32)
    w1s = lax.dot_general(z1_ref[:, 0:G], dis, dn, precision=hi,
                          preferred_element_type=f32)
    w2s = lax.dot_general(z2_ref[:, 0:G], dis, dn, precision=hi,
                          preferred_element_type=f32)
    mm = functools.partial(lax.dot_general,
                           dimension_numbers=(((1,), (0,)), ((), ())),
                           precision=hi, preferred_element_type=f32)
    t1 = mm(w3_ref[...], wfc_ref[...])
    t2 = mm(w2_ref[...], t1)
    m3 = mm(w1_ref[...], t2)
    c2 = mm(b1_ref[...], t2)
    c1 = mm(b2_ref[...], t1)
    c0 = mm(b3_ref[...], wfc_ref[...])
    raw = mm(q, m3) + mm(w2s, c2) + mm(w1s, c1) + mm(cnt, c0)
    o_ref[...] = raw / jnp.maximum(cnt, 1.0) + bfc_ref[...]


def _head(y0, z1, z2, p3, dis, xpad, W_in, W_h0, W_h1, W_fc,
          b_in, b_h0, b_h1, b_fc):
    return pl.pallas_call(
        _head_body,
        out_shape=jax.ShapeDtypeStruct((G, G), jnp.float32),
    )(y0, z1, z2, p3, dis, xpad, W_in, W_h0, W_h1, W_fc,
      b_in.reshape(1, D), b_h0.reshape(1, D), b_h1.reshape(1, D),
      b_fc.reshape(1, G))


def kernel(x, edge_index, batch, W_in, b_in, W_h0, b_h0, W_h1, b_h1, W_fc, b_fc):
    i32 = jnp.int32
    src = edge_index[0].astype(i32)
    dst = edge_index[1].astype(i32)
    loops = jnp.arange(N, dtype=i32)
    padi = jnp.full((EP - E_LOOP,), BIN, i32)
    s_p = jnp.concatenate([src, loops, padi]).reshape(NW, CHK, B)
    d_p = jnp.concatenate([dst, loops, padi]).reshape(NW, CHK, B)
    bpad = jnp.pad(batch.astype(i32), (0, NPAD - N), constant_values=-1)
    xpad = jnp.pad(x, ((0, NPAD - N), (0, 0)))

    degpart = _k_deg(d_p)
    y0 = _k_onehot(bpad).reshape(NPAD, W128)
    y0s, dis, dinv = _prep(degpart, y0)
    p1 = _k_pass(y0s, s_p, d_p)
    z1, y1s = _comb(p1, dinv)
    p2 = _k_pass(y1s, s_p, d_p)
    z2, y2s = _comb(p2, dinv)
    p3 = _k_pass(y2s, s_p, d_p)
    return _head(y0, z1, z2, p3, dis, xpad, W_in, W_h0, W_h1, W_fc,
                 b_in, b_h0, b_h1, b_fc)
